# baseline (device time: 10945 ns/iter reference)
import jax
import jax.numpy as jnp
from jax import lax
from jax.experimental import pallas as pl
from jax.experimental.pallas import tpu as pltpu

N_DEV = 4
N_CHUNK = 4


def kernel(x):
    m, n = x.shape
    cm = m // N_CHUNK

    def body(x_hbm, out_hbm, e_buf, stats_ref, in_sems, out_sems,
             send_sems, recv_sems):
        my = lax.axis_index("i")

        in_copies = []
        for c in range(N_CHUNK):
            in_copies.append(pltpu.make_async_copy(
                x_hbm.at[pl.ds(c * cm, cm), :], e_buf.at[c], in_sems.at[c],
            ))
        in_copies[0].start()

        barrier_sem = pltpu.get_barrier_semaphore()
        for r in range(1, N_DEV):
            pl.semaphore_signal(
                barrier_sem, inc=1,
                device_id=((my + r) % N_DEV,),
                device_id_type=pl.DeviceIdType.MESH,
            )

        rdmas = [[None] * N_DEV for _ in range(N_CHUNK)]

        for c in range(N_CHUNK):
            if c + 1 < N_CHUNK:
                in_copies[c + 1].start()
            in_copies[c].wait()
            e = jnp.exp(e_buf[c])
            e_buf[c] = e
            ls = jnp.sum(e, axis=1, keepdims=True)
            stats_ref[c, 0, :, :] = jnp.transpose(ls, (1, 0))
            if c == 0:
                pl.semaphore_wait(barrier_sem, N_DEV - 1)
            for r in range(1, N_DEV):
                rdma = pltpu.make_async_remote_copy(
                    src_ref=stats_ref.at[c, 0],
                    dst_ref=stats_ref.at[c, N_DEV - r],
                    send_sem=send_sems.at[c, r],
                    recv_sem=recv_sems.at[c, N_DEV - r],
                    device_id=((my + r) % N_DEV,),
                    device_id_type=pl.DeviceIdType.MESH,
                )
                rdma.start()
                rdmas[c][r] = rdma

        out_copies = []
        for c in range(N_CHUNK):
            for r in range(1, N_DEV):
                rdmas[c][r].wait()
            gs = jnp.sum(stats_ref[c, :, :, :], axis=0)
            scale = jnp.transpose(1.0 / gs, (1, 0))
            e_buf[c] = e_buf[c] * scale
            cp = pltpu.make_async_copy(
                e_buf.at[c], out_hbm.at[pl.ds(c * cm, cm), :], out_sems.at[c],
            )
            cp.start()
            out_copies.append(cp)
        for cp in out_copies:
            cp.wait()

    return pl.pallas_call(
        body,
        out_shape=jax.ShapeDtypeStruct((m, n), x.dtype),
        in_specs=[pl.BlockSpec(memory_space=pl.ANY)],
        out_specs=pl.BlockSpec(memory_space=pl.ANY),
        scratch_shapes=[
            pltpu.VMEM((N_CHUNK, cm, n), jnp.float32),
            pltpu.VMEM((N_CHUNK, N_DEV, 1, cm), jnp.float32),
            pltpu.SemaphoreType.DMA((N_CHUNK,)),
            pltpu.SemaphoreType.DMA((N_CHUNK,)),
            pltpu.SemaphoreType.DMA((N_CHUNK, N_DEV)),
            pltpu.SemaphoreType.DMA((N_CHUNK, N_DEV)),
        ],
        compiler_params=pltpu.CompilerParams(collective_id=0),
    )(x)


# device time: 10096 ns/iter; 1.0841x vs baseline; 1.0841x over previous
import jax
import jax.numpy as jnp
from jax import lax
from jax.experimental import pallas as pl
from jax.experimental.pallas import tpu as pltpu

N_DEV = 4


def kernel(x):
    m, n = x.shape

    def body(x_ref, out_ref, stats_ref, send_sems, recv_sems):
        my = lax.axis_index("i")

        barrier_sem = pltpu.get_barrier_semaphore()
        for r in range(1, N_DEV):
            pl.semaphore_signal(
                barrier_sem, inc=1,
                device_id=((my + r) % N_DEV,),
                device_id_type=pl.DeviceIdType.MESH,
            )

        e = jnp.exp(x_ref[:, :])
        out_ref[:, :] = e
        ls = jnp.sum(e, axis=1, keepdims=True)
        stats_ref[0, :, :] = jnp.transpose(ls, (1, 0))

        pl.semaphore_wait(barrier_sem, N_DEV - 1)

        rdmas = []
        for r in range(1, N_DEV):
            rdma = pltpu.make_async_remote_copy(
                src_ref=stats_ref.at[0],
                dst_ref=stats_ref.at[N_DEV - r],
                send_sem=send_sems.at[r],
                recv_sem=recv_sems.at[N_DEV - r],
                device_id=((my + r) % N_DEV,),
                device_id_type=pl.DeviceIdType.MESH,
            )
            rdma.start()
            rdmas.append(rdma)
        for rdma in rdmas:
            rdma.wait()

        gs = jnp.sum(stats_ref[:, :, :], axis=0)
        out_ref[:, :] = out_ref[:, :] * jnp.transpose(1.0 / gs, (1, 0))

    return pl.pallas_call(
        body,
        out_shape=jax.ShapeDtypeStruct((m, n), x.dtype),
        in_specs=[pl.BlockSpec(memory_space=pltpu.VMEM)],
        out_specs=pl.BlockSpec(memory_space=pltpu.VMEM),
        scratch_shapes=[
            pltpu.VMEM((N_DEV, 1, m), jnp.float32),
            pltpu.SemaphoreType.DMA((N_DEV,)),
            pltpu.SemaphoreType.DMA((N_DEV,)),
        ],
        compiler_params=pltpu.CompilerParams(collective_id=0),
    )(x)
